# Initial kernel scaffold; baseline (speedup 1.0000x reference)
import jax
import jax.numpy as jnp
from jax.experimental import pallas as pl


def kernel(user_ids, item_ids, feature_ids, user_table, item_table, feature_table):
    raise NotImplementedError("write your pallas kernel here")


# speedup vs baseline: 3.2137x; 3.2137x over previous
import functools

import jax
import jax.numpy as jnp
from jax import lax
from jax.experimental import pallas as pl
from jax.experimental.pallas import tpu as pltpu
from jax.experimental.pallas import tpu_sc as plsc

NUM_USERS = 1000000
NUM_ITEMS = 100000
NUM_FEATURES = 100000
EMBED_DIM = 128
BATCH = 16384
N_FIELDS = 26

NC = 2
NS = 16
NW = NC * NS

CHUNK = 512


def _lookup_kernel(user_ids, item_ids, feature_ids_flat,
                   user_table, item_table, feature_table):
    per_w_u = BATCH // NW
    per_w_f = (BATCH * N_FIELDS) // NW
    mesh = plsc.VectorSubcoreMesh(core_axis_name="c", subcore_axis_name="s")

    @functools.partial(
        pl.kernel,
        mesh=mesh,
        out_type=(
            jax.ShapeDtypeStruct((BATCH, EMBED_DIM), jnp.float32),
            jax.ShapeDtypeStruct((BATCH, EMBED_DIM), jnp.float32),
            jax.ShapeDtypeStruct((BATCH * N_FIELDS, EMBED_DIM), jnp.float32),
        ),
        scratch_types=[
            pltpu.VMEM((CHUNK,), jnp.int32),
            pltpu.VMEM((CHUNK, EMBED_DIM), jnp.float32),
            pltpu.SemaphoreType.DMA,
        ],
    )
    def k(uids, iids, fids, utab, itab, ftab, uout, iout, fout,
          idx_v, rows_v, sem):
        wid = lax.axis_index("s") * NC + lax.axis_index("c")

        def phase(ids_hbm, tab_hbm, out_hbm, per_w):
            wbase = wid * per_w
            nch = per_w // CHUNK

            def body(c, carry):
                base = wbase + c * CHUNK
                pltpu.sync_copy(ids_hbm.at[pl.ds(base, CHUNK)], idx_v)
                pltpu.async_copy(tab_hbm.at[idx_v], rows_v, sem).wait()
                pltpu.sync_copy(rows_v, out_hbm.at[pl.ds(base, CHUNK)])
                return carry

            lax.fori_loop(0, nch, body, 0)

        phase(uids, utab, uout, per_w_u)
        phase(iids, itab, iout, per_w_u)
        phase(fids, ftab, fout, per_w_f)

    return k(user_ids, item_ids, feature_ids_flat,
             user_table, item_table, feature_table)


def kernel(user_ids, item_ids, feature_ids, user_table, item_table,
           feature_table):
    user_emb, item_emb, feat_flat = _lookup_kernel(
        user_ids, item_ids, feature_ids.reshape(-1),
        user_table, item_table, feature_table)
    return (user_emb, item_emb,
            feat_flat.reshape(BATCH, N_FIELDS, EMBED_DIM))


# speedup vs baseline: 3.3351x; 1.0378x over previous
import functools

import jax
import jax.numpy as jnp
from jax import lax
from jax.experimental import pallas as pl
from jax.experimental.pallas import tpu as pltpu
from jax.experimental.pallas import tpu_sc as plsc

NUM_USERS = 1000000
NUM_ITEMS = 100000
NUM_FEATURES = 100000
EMBED_DIM = 128
BATCH = 16384
N_FIELDS = 26

NC = 2
NS = 16
NW = NC * NS

CHUNK = 256
NBUF = 2
IDX_MAX = (BATCH * N_FIELDS) // NW


def _lookup_kernel(user_ids, item_ids, feature_ids_flat,
                   user_table, item_table, feature_table):
    per_w_u = BATCH // NW
    per_w_f = (BATCH * N_FIELDS) // NW
    mesh = plsc.VectorSubcoreMesh(core_axis_name="c", subcore_axis_name="s")

    @functools.partial(
        pl.kernel,
        mesh=mesh,
        out_type=(
            jax.ShapeDtypeStruct((BATCH, EMBED_DIM), jnp.float32),
            jax.ShapeDtypeStruct((BATCH, EMBED_DIM), jnp.float32),
            jax.ShapeDtypeStruct((BATCH * N_FIELDS, EMBED_DIM), jnp.float32),
        ),
        scratch_types=[
            pltpu.VMEM((IDX_MAX,), jnp.int32),
            pltpu.VMEM((CHUNK, EMBED_DIM), jnp.float32),
            pltpu.VMEM((CHUNK, EMBED_DIM), jnp.float32),
            pltpu.SemaphoreType.DMA,
            pltpu.SemaphoreType.DMA,
            pltpu.SemaphoreType.DMA,
            pltpu.SemaphoreType.DMA,
        ],
    )
    def k(uids, iids, fids, utab, itab, ftab, uout, iout, fout,
          idx_v, rows0, rows1, g0, g1, o0, o1):
        wid = lax.axis_index("s") * NC + lax.axis_index("c")
        rows = (rows0, rows1)
        gsem = (g0, g1)
        osem = (o0, o1)

        def phase(ids_hbm, tab_hbm, out_hbm, per_w):
            wbase = wid * per_w
            nch = per_w // CHUNK
            npair = nch // NBUF
            pltpu.sync_copy(ids_hbm.at[pl.ds(wbase, per_w)],
                            idx_v.at[pl.ds(0, per_w)])

            def idx_at(c):
                return idx_v.at[pl.ds(c * CHUNK, CHUNK)]

            def out_at(c):
                return out_hbm.at[pl.ds(wbase + c * CHUNK, CHUNK)]

            for b in range(NBUF):
                pltpu.async_copy(tab_hbm.at[idx_at(b)], rows[b], gsem[b])

            def body(g, carry):
                for b in range(NBUF):
                    c = g * NBUF + b
                    pltpu.make_async_copy(tab_hbm.at[idx_at(c)], rows[b],
                                          gsem[b]).wait()
                    pltpu.async_copy(rows[b], out_at(c), osem[b])
                    pltpu.make_async_copy(rows[b], out_at(c), osem[b]).wait()
                    pltpu.async_copy(tab_hbm.at[idx_at(c + NBUF)], rows[b],
                                     gsem[b])
                return carry

            lax.fori_loop(0, npair - 1, body, 0)

            for b in range(NBUF):
                c = nch - NBUF + b
                pltpu.make_async_copy(tab_hbm.at[idx_at(c)], rows[b],
                                      gsem[b]).wait()
                pltpu.async_copy(rows[b], out_at(c), osem[b])
            for b in range(NBUF):
                c = nch - NBUF + b
                pltpu.make_async_copy(rows[b], out_at(c), osem[b]).wait()

        phase(uids, utab, uout, per_w_u)
        phase(iids, itab, iout, per_w_u)
        phase(fids, ftab, fout, per_w_f)

    return k(user_ids, item_ids, feature_ids_flat,
             user_table, item_table, feature_table)


def kernel(user_ids, item_ids, feature_ids, user_table, item_table,
           feature_table):
    user_emb, item_emb, feat_flat = _lookup_kernel(
        user_ids, item_ids, feature_ids.reshape(-1),
        user_table, item_table, feature_table)
    return (user_emb, item_emb,
            feat_flat.reshape(BATCH, N_FIELDS, EMBED_DIM))


# speedup vs baseline: 5.4845x; 1.6445x over previous
import functools

import jax
import jax.numpy as jnp
from jax import lax
from jax.experimental import pallas as pl
from jax.experimental.pallas import tpu as pltpu
from jax.experimental.pallas import tpu_sc as plsc

NUM_USERS = 1000000
NUM_ITEMS = 100000
NUM_FEATURES = 100000
EMBED_DIM = 128
BATCH = 16384
N_FIELDS = 26

NC = 2
NS = 16
NW = NC * NS

CHUNK = 256
FCB = 16
FROWS = FCB * N_FIELDS
NBUF = 2
IDX_MAX = (BATCH * N_FIELDS) // NW


def _lookup_kernel(user_ids, item_ids, feature_ids_flat,
                   user_table, item_table, feature_table):
    per_w_u = BATCH // NW
    per_w_f = (BATCH * N_FIELDS) // NW
    bat_w = BATCH // NW
    mesh = plsc.VectorSubcoreMesh(core_axis_name="c", subcore_axis_name="s")

    @functools.partial(
        pl.kernel,
        mesh=mesh,
        out_type=(
            jax.ShapeDtypeStruct((BATCH, EMBED_DIM), jnp.float32),
            jax.ShapeDtypeStruct((BATCH, EMBED_DIM), jnp.float32),
            jax.ShapeDtypeStruct((BATCH, N_FIELDS, EMBED_DIM), jnp.float32),
        ),
        scratch_types=[
            pltpu.VMEM((IDX_MAX,), jnp.int32),
            pltpu.VMEM((FROWS, EMBED_DIM), jnp.float32),
            pltpu.VMEM((FROWS, EMBED_DIM), jnp.float32),
            pltpu.SemaphoreType.DMA,
            pltpu.SemaphoreType.DMA,
            pltpu.SemaphoreType.DMA,
            pltpu.SemaphoreType.DMA,
        ],
    )
    def k(uids, iids, fids, utab, itab, ftab, uout, iout, fout,
          idx_v, rows0, rows1, g0, g1, o0, o1):
        wid = lax.axis_index("s") * NC + lax.axis_index("c")
        rows = (rows0, rows1)
        gsem = (g0, g1)
        osem = (o0, o1)

        def phase(ids_hbm, tab_hbm, out_hbm, per_w):
            wbase = wid * per_w
            nch = per_w // CHUNK
            npair = nch // NBUF
            pltpu.sync_copy(ids_hbm.at[pl.ds(wbase, per_w)],
                            idx_v.at[pl.ds(0, per_w)])

            def idx_at(c):
                return idx_v.at[pl.ds(c * CHUNK, CHUNK)]

            def buf_at(b):
                return rows[b].at[pl.ds(0, CHUNK)]

            def out_at(c):
                return out_hbm.at[pl.ds(wbase + c * CHUNK, CHUNK)]

            for b in range(NBUF):
                pltpu.async_copy(tab_hbm.at[idx_at(b)], buf_at(b), gsem[b])

            def body(g, carry):
                for b in range(NBUF):
                    c = g * NBUF + b
                    pltpu.make_async_copy(tab_hbm.at[idx_at(c)], buf_at(b),
                                          gsem[b]).wait()
                    pltpu.async_copy(buf_at(b), out_at(c), osem[b])
                    pltpu.make_async_copy(buf_at(b), out_at(c), osem[b]).wait()
                    pltpu.async_copy(tab_hbm.at[idx_at(c + NBUF)], buf_at(b),
                                     gsem[b])
                return carry

            lax.fori_loop(0, npair - 1, body, 0)

            for b in range(NBUF):
                c = nch - NBUF + b
                pltpu.make_async_copy(tab_hbm.at[idx_at(c)], buf_at(b),
                                      gsem[b]).wait()
                pltpu.async_copy(buf_at(b), out_at(c), osem[b])
            for b in range(NBUF):
                c = nch - NBUF + b
                pltpu.make_async_copy(buf_at(b), out_at(c), osem[b]).wait()

        def fphase(ids_hbm, tab_hbm, out_hbm):
            wbase = wid * per_w_f
            bbase = wid * bat_w
            nch = bat_w // FCB
            npair = nch // NBUF
            pltpu.sync_copy(ids_hbm.at[pl.ds(wbase, per_w_f)], idx_v)

            def idx_at(c):
                return idx_v.at[pl.ds(c * FROWS, FROWS)]

            def start_out(b, c):
                for i in range(FCB):
                    pltpu.async_copy(
                        rows[b].at[pl.ds(i * N_FIELDS, N_FIELDS)],
                        out_hbm.at[bbase + c * FCB + i], osem[b])

            def wait_out(b, c):
                for i in range(FCB):
                    pltpu.make_async_copy(
                        rows[b].at[pl.ds(i * N_FIELDS, N_FIELDS)],
                        out_hbm.at[bbase + c * FCB + i], osem[b]).wait()

            for b in range(NBUF):
                pltpu.async_copy(tab_hbm.at[idx_at(b)], rows[b], gsem[b])

            def body(g, carry):
                for b in range(NBUF):
                    c = g * NBUF + b
                    pltpu.make_async_copy(tab_hbm.at[idx_at(c)], rows[b],
                                          gsem[b]).wait()
                    start_out(b, c)
                    wait_out(b, c)
                    pltpu.async_copy(tab_hbm.at[idx_at(c + NBUF)], rows[b],
                                     gsem[b])
                return carry

            lax.fori_loop(0, npair - 1, body, 0)

            for b in range(NBUF):
                c = nch - NBUF + b
                pltpu.make_async_copy(tab_hbm.at[idx_at(c)], rows[b],
                                      gsem[b]).wait()
                start_out(b, c)
            for b in range(NBUF):
                wait_out(b, nch - NBUF + b)

        phase(uids, utab, uout, per_w_u)
        phase(iids, itab, iout, per_w_u)
        fphase(fids, ftab, fout)

    return k(user_ids, item_ids, feature_ids_flat,
             user_table, item_table, feature_table)


def kernel(user_ids, item_ids, feature_ids, user_table, item_table,
           feature_table):
    return _lookup_kernel(
        user_ids, item_ids, feature_ids.reshape(-1),
        user_table, item_table, feature_table)


# speedup vs baseline: 11.1946x; 2.0411x over previous
import functools

import jax
import jax.numpy as jnp
from jax import lax
from jax.experimental import pallas as pl
from jax.experimental.pallas import tpu as pltpu
from jax.experimental.pallas import tpu_sc as plsc

NUM_USERS = 1000000
NUM_ITEMS = 100000
NUM_FEATURES = 100000
EMBED_DIM = 128
BATCH = 16384
N_FIELDS = 26

NC = 2
NS = 16
NW = NC * NS

CHUNK = 256
NBUF = 2
BAT_W = BATCH // NW
HALVES = BAT_W // CHUNK


def _lookup_kernel(user_ids, item_ids, feature_ids_t,
                   user_table, item_table, feature_table):
    mesh = plsc.VectorSubcoreMesh(core_axis_name="c", subcore_axis_name="s")

    @functools.partial(
        pl.kernel,
        mesh=mesh,
        out_type=(
            jax.ShapeDtypeStruct((BATCH, EMBED_DIM), jnp.float32),
            jax.ShapeDtypeStruct((BATCH, EMBED_DIM), jnp.float32),
            jax.ShapeDtypeStruct((N_FIELDS * BATCH, EMBED_DIM), jnp.float32),
        ),
        scratch_types=[
            pltpu.VMEM((N_FIELDS * BAT_W,), jnp.int32),
            pltpu.VMEM((CHUNK, EMBED_DIM), jnp.float32),
            pltpu.VMEM((CHUNK, EMBED_DIM), jnp.float32),
            pltpu.SemaphoreType.DMA,
            pltpu.SemaphoreType.DMA,
            pltpu.SemaphoreType.DMA,
            pltpu.SemaphoreType.DMA,
        ],
    )
    def k(uids, iids, fids_t, utab, itab, ftab, uout, iout, fout,
          idx_v, rows0, rows1, g0, g1, o0, o1):
        wid = lax.axis_index("s") * NC + lax.axis_index("c")
        rows = (rows0, rows1)
        gsem = (g0, g1)
        osem = (o0, o1)

        def run_ring(nch, idx_at, out_at):
            npair = nch // NBUF

            for b in range(NBUF):
                pltpu.async_copy(ftab_or(idx_at, b), rows[b], gsem[b])

            def body(g, carry):
                for b in range(NBUF):
                    c = g * NBUF + b
                    pltpu.make_async_copy(ftab_or(idx_at, c), rows[b],
                                          gsem[b]).wait()
                    pltpu.async_copy(rows[b], out_at(c), osem[b])
                    pltpu.make_async_copy(rows[b], out_at(c), osem[b]).wait()
                    pltpu.async_copy(ftab_or(idx_at, c + NBUF), rows[b],
                                     gsem[b])
                return carry

            lax.fori_loop(0, npair - 1, body, 0)

            for b in range(NBUF):
                c = nch - NBUF + b
                pltpu.make_async_copy(ftab_or(idx_at, c), rows[b],
                                      gsem[b]).wait()
                pltpu.async_copy(rows[b], out_at(c), osem[b])
            for b in range(NBUF):
                pltpu.make_async_copy(rows[b], out_at(nch - NBUF + b),
                                      osem[b]).wait()

        def ftab_or(idx_at, c):
            tab, idx = idx_at(c)
            return tab.at[idx]

        def phase1d(ids_hbm, tab_hbm, out_hbm):
            wbase = wid * BAT_W
            pltpu.sync_copy(ids_hbm.at[pl.ds(wbase, BAT_W)],
                            idx_v.at[pl.ds(0, BAT_W)])

            def idx_at(c):
                return tab_hbm, idx_v.at[pl.ds(c * CHUNK, CHUNK)]

            def out_at(c):
                return out_hbm.at[pl.ds(wbase + c * CHUNK, CHUNK)]

            run_ring(BAT_W // CHUNK, idx_at, out_at)

        def fphase(ids2_hbm, tab_hbm, out_hbm):
            bbase = wid * BAT_W
            for f in range(N_FIELDS):
                pltpu.async_copy(ids2_hbm.at[f, pl.ds(bbase, BAT_W)],
                                 idx_v.at[pl.ds(f * BAT_W, BAT_W)], o0)
            for f in range(N_FIELDS):
                pltpu.make_async_copy(
                    ids2_hbm.at[f, pl.ds(bbase, BAT_W)],
                    idx_v.at[pl.ds(f * BAT_W, BAT_W)], o0).wait()

            def idx_at(c):
                return tab_hbm, idx_v.at[pl.ds(c * CHUNK, CHUNK)]

            def out_at(c):
                f = c // HALVES
                h = c % HALVES
                return out_hbm.at[
                    pl.ds(f * BATCH + bbase + h * CHUNK, CHUNK)]

            run_ring(N_FIELDS * HALVES, idx_at, out_at)

        phase1d(uids, utab, uout)
        phase1d(iids, itab, iout)
        fphase(fids_t, ftab, fout)

    return k(user_ids, item_ids, feature_ids_t,
             user_table, item_table, feature_table)


def kernel(user_ids, item_ids, feature_ids, user_table, item_table,
           feature_table):
    user_emb, item_emb, feat_fmajor = _lookup_kernel(
        user_ids, item_ids, jnp.transpose(feature_ids),
        user_table, item_table, feature_table)
    feat_emb = feat_fmajor.reshape(N_FIELDS, BATCH, EMBED_DIM)
    return (user_emb, item_emb, jnp.transpose(feat_emb, (1, 0, 2)))


# speedup vs baseline: 11.2536x; 1.0053x over previous
import functools

import jax
import jax.numpy as jnp
from jax import lax
from jax.experimental import pallas as pl
from jax.experimental.pallas import tpu as pltpu
from jax.experimental.pallas import tpu_sc as plsc

NUM_USERS = 1000000
NUM_ITEMS = 100000
NUM_FEATURES = 100000
EMBED_DIM = 128
BATCH = 16384
N_FIELDS = 26

NC = 2
NS = 16
NW = NC * NS

CHUNK = 256
NBUF = 2
BAT_W = BATCH // NW
HALVES = BAT_W // CHUNK
U_OFF = 0
I_OFF = BAT_W
F_OFF = 2 * BAT_W
IDX_WORDS = (2 + N_FIELDS) * BAT_W


def _lookup_kernel(user_ids, item_ids, feature_ids_t,
                   user_table, item_table, feature_table):
    mesh = plsc.VectorSubcoreMesh(core_axis_name="c", subcore_axis_name="s")

    @functools.partial(
        pl.kernel,
        mesh=mesh,
        out_type=(
            jax.ShapeDtypeStruct((BATCH, EMBED_DIM), jnp.float32),
            jax.ShapeDtypeStruct((BATCH, EMBED_DIM), jnp.float32),
            jax.ShapeDtypeStruct((N_FIELDS * BATCH, EMBED_DIM), jnp.float32),
        ),
        scratch_types=[
            pltpu.VMEM((IDX_WORDS,), jnp.int32),
            pltpu.VMEM((CHUNK, EMBED_DIM), jnp.float32),
            pltpu.VMEM((CHUNK, EMBED_DIM), jnp.float32),
            pltpu.SemaphoreType.DMA,
            pltpu.SemaphoreType.DMA,
            pltpu.SemaphoreType.DMA,
            pltpu.SemaphoreType.DMA,
            pltpu.SemaphoreType.DMA,
        ],
    )
    def k(uids, iids, fids_t, utab, itab, ftab, uout, iout, fout,
          idx_v, rows0, rows1, g0, g1, o0, o1, psem):
        wid = lax.axis_index("s") * NC + lax.axis_index("c")
        rows = (rows0, rows1)
        gsem = (g0, g1)
        osem = (o0, o1)
        wbase = wid * BAT_W

        def pre_descs():
            descs = [
                (uids.at[pl.ds(wbase, BAT_W)],
                 idx_v.at[pl.ds(U_OFF, BAT_W)]),
                (iids.at[pl.ds(wbase, BAT_W)],
                 idx_v.at[pl.ds(I_OFF, BAT_W)]),
            ]
            for f in range(N_FIELDS):
                descs.append((fids_t.at[f, pl.ds(wbase, BAT_W)],
                              idx_v.at[pl.ds(F_OFF + f * BAT_W, BAT_W)]))
            return descs

        for src, dst in pre_descs():
            pltpu.async_copy(src, dst, psem)

        def wait_prefetch(n_slices):
            src, dst = pre_descs()[0]
            for _ in range(n_slices):
                pltpu.make_async_copy(src, dst, psem).wait()

        def run_ring(nch, idx_at, tab_hbm, out_at):
            npair = nch // NBUF

            for b in range(NBUF):
                pltpu.async_copy(tab_hbm.at[idx_at(b)], rows[b], gsem[b])

            def body(g, carry):
                for b in range(NBUF):
                    c = g * NBUF + b
                    pltpu.make_async_copy(tab_hbm.at[idx_at(c)], rows[b],
                                          gsem[b]).wait()
                    pltpu.async_copy(rows[b], out_at(c), osem[b])
                    pltpu.make_async_copy(rows[b], out_at(c), osem[b]).wait()
                    pltpu.async_copy(tab_hbm.at[idx_at(c + NBUF)], rows[b],
                                     gsem[b])
                return carry

            lax.fori_loop(0, npair - 1, body, 0)

            for b in range(NBUF):
                c = nch - NBUF + b
                pltpu.make_async_copy(tab_hbm.at[idx_at(c)], rows[b],
                                      gsem[b]).wait()
                pltpu.async_copy(rows[b], out_at(c), osem[b])
            for b in range(NBUF):
                pltpu.make_async_copy(rows[b], out_at(nch - NBUF + b),
                                      osem[b]).wait()

        def phase1d(off, tab_hbm, out_hbm):
            def idx_at(c):
                return idx_v.at[pl.ds(off + c * CHUNK, CHUNK)]

            def out_at(c):
                return out_hbm.at[pl.ds(wbase + c * CHUNK, CHUNK)]

            run_ring(BAT_W // CHUNK, idx_at, tab_hbm, out_at)

        def fphase(tab_hbm, out_hbm):
            def idx_at(c):
                return idx_v.at[pl.ds(F_OFF + c * CHUNK, CHUNK)]

            def out_at(c):
                f = c // HALVES
                h = c % HALVES
                return out_hbm.at[
                    pl.ds(f * BATCH + wbase + h * CHUNK, CHUNK)]

            run_ring(N_FIELDS * HALVES, idx_at, tab_hbm, out_at)

        wait_prefetch(2)
        phase1d(U_OFF, utab, uout)
        phase1d(I_OFF, itab, iout)
        wait_prefetch(N_FIELDS)
        fphase(ftab, fout)

    return k(user_ids, item_ids, feature_ids_t,
             user_table, item_table, feature_table)


def kernel(user_ids, item_ids, feature_ids, user_table, item_table,
           feature_table):
    user_emb, item_emb, feat_fmajor = _lookup_kernel(
        user_ids, item_ids, jnp.transpose(feature_ids),
        user_table, item_table, feature_table)
    feat_emb = feat_fmajor.reshape(N_FIELDS, BATCH, EMBED_DIM)
    return (user_emb, item_emb, jnp.transpose(feat_emb, (1, 0, 2)))
